# Initial kernel scaffold; baseline (speedup 1.0000x reference)
import jax
import jax.numpy as jnp
from jax.experimental import pallas as pl


def kernel(x, edge_index, W1, b1, W2, b2, W3, b3, g1, be1, g2, be2, g3, be3):
    raise NotImplementedError("write your pallas kernel here")


# speedup vs baseline: 28.9655x; 28.9655x over previous
import functools

import jax
import jax.numpy as jnp
from jax import lax
from jax.experimental import pallas as pl
from jax.experimental.pallas import tpu as pltpu
from jax.experimental.pallas import tpu_sc as plsc

_L = 1024
_E = 16384
_G = 32
_B = 4
_N = 8
_C = 128
_NTILES = 32
_ROWS = _L // _NTILES
_SENT = 2 ** 30


def _sc_count_body(src_hbm, dst_hbm, c_hbm, src_v, dst_v, loc_v):
    wid = lax.axis_index("s") * 2 + lax.axis_index("c")
    base = wid * _ROWS

    pltpu.sync_copy(src_hbm, src_v)
    pltpu.sync_copy(dst_hbm, dst_v)

    zer = jnp.zeros((16,), jnp.float32)

    def zrow(r, carry):
        def zcol(j, carry2):
            loc_v[r, pl.ds(pl.multiple_of(j * 16, 16), 16)] = zer
            return carry2
        return lax.fori_loop(0, _L // 16, zcol, carry)
    lax.fori_loop(0, _ROWS, zrow, 0)

    lane = lax.iota(jnp.int32, 16)
    ones = jnp.ones((16,), jnp.float32)

    def edge(i, carry):
        off = pl.multiple_of(i * 16, 16)
        s = src_v[pl.ds(off, 16)]
        d = dst_v[pl.ds(off, 16)]
        r = d - base
        ok = (r >= 0) & (r < _ROWS)
        key = jnp.where(ok, r * _L + s, _SENT)
        ks, vs = plsc.sort_key_val(key, ones)
        for t in (1, 2, 4, 8):
            prev = jnp.maximum(lane - t, 0)
            kp = ks.at[prev].get(mode="promise_in_bounds")
            vp = vs.at[prev].get(mode="promise_in_bounds")
            vs = vs + jnp.where((lane >= t) & (kp == ks), vp, 0.0)
        nxt = ks.at[jnp.minimum(lane + 1, 15)].get(mode="promise_in_bounds")
        last = (ks != nxt) | (lane == 15)
        m = last & (ks < _ROWS * _L)
        row = jnp.where(m, ks >> 10, 0)
        col = jnp.where(m, ks & (_L - 1), 0)
        plsc.addupdate_scatter(loc_v, [row, col], vs, mask=m)
        return carry

    lax.fori_loop(0, _E // 16, edge, 0)
    pltpu.sync_copy(loc_v, c_hbm.at[pl.ds(base, _ROWS)])


@functools.cache
def _sc_count_call():
    mesh = plsc.VectorSubcoreMesh(
        core_axis_name="c", subcore_axis_name="s", num_cores=2, num_subcores=16)
    return pl.kernel(
        _sc_count_body,
        out_type=jax.ShapeDtypeStruct((_L, _L), jnp.float32),
        mesh=mesh,
        compiler_params=pltpu.CompilerParams(needs_layout_passes=False),
        scratch_types=[
            pltpu.VMEM((_E,), jnp.int32),
            pltpu.VMEM((_E,), jnp.int32),
            pltpu.VMEM((_ROWS, _L), jnp.float32),
        ],
    )


def _deg_body(c_ref, col_ref, row_ref):
    cm = c_ref[...]
    ones_col = jnp.ones((_L, 1), jnp.float32)
    deg_col = jnp.dot(cm, ones_col, preferred_element_type=jnp.float32) + 2.0
    dis_col = lax.rsqrt(deg_col)
    nl_col = 2.0 / deg_col
    col_ref[...] = jnp.concatenate([dis_col, nl_col], axis=1)
    ones_row = jnp.ones((1, _L), jnp.float32)
    deg_row = lax.dot_general(ones_row, cm, (((1,), (1,)), ((), ())),
                              preferred_element_type=jnp.float32) + 2.0
    dis_row = lax.rsqrt(deg_row)
    nl_row = 2.0 / deg_row
    row_ref[...] = jnp.concatenate([dis_row, nl_row], axis=0)


_deg_call = pl.pallas_call(
    _deg_body,
    out_shape=[
        jax.ShapeDtypeStruct((_L, 2), jnp.float32),
        jax.ShapeDtypeStruct((2, _L), jnp.float32),
    ],
)


def _xw_body(x_ref, w_ref, o_ref):
    o_ref[0] = lax.dot_general(x_ref[0], w_ref[...], (((0,), (0,)), ((), ())),
                               preferred_element_type=jnp.float32)


_xw_call = pl.pallas_call(
    _xw_body,
    grid=(_G,),
    in_specs=[
        pl.BlockSpec((1, _C, _L), lambda i: (i, 0, 0)),
        pl.BlockSpec((_C, _C), lambda i: (0, 0)),
    ],
    out_specs=pl.BlockSpec((1, _L, _C), lambda i: (i, 0, 0)),
    out_shape=jax.ShapeDtypeStruct((_G, _L, _C), jnp.float32),
)


def _conv_body(t_ref, c_ref, dn_ref, b_ref, y_ref, xm_ref, st_ref):
    i = pl.program_id(0)
    n = lax.rem(i, _N)
    t = t_ref[0]
    dis = dn_ref[:, 0:1]
    nl = dn_ref[:, 1:2]
    v = jnp.dot(c_ref[...], t * dis, preferred_element_type=jnp.float32)
    y = v * dis + t * nl + b_ref[...]
    y_ref[0] = y

    @pl.when(i == 0)
    def _():
        st_ref[...] = jnp.zeros((4, _C), jnp.float32)

    st_ref[0:1] = st_ref[0:1] + jnp.sum(y, axis=0, keepdims=True)
    st_ref[1:2] = st_ref[1:2] + jnp.sum(y * y, axis=0, keepdims=True)

    @pl.when(n == 0)
    def _():
        xm_ref[0] = y

    @pl.when(n != 0)
    def _():
        xm_ref[0] = jnp.maximum(xm_ref[0], y)

    @pl.when(n == _N - 1)
    def _():
        m = xm_ref[0]
        st_ref[2:3] = st_ref[2:3] + jnp.sum(m, axis=0, keepdims=True)
        st_ref[3:4] = st_ref[3:4] + jnp.sum(m * m, axis=0, keepdims=True)


_conv_call = pl.pallas_call(
    _conv_body,
    grid=(_G,),
    in_specs=[
        pl.BlockSpec((1, _L, _C), lambda i: (i, 0, 0)),
        pl.BlockSpec((_L, _L), lambda i: (0, 0)),
        pl.BlockSpec((_L, 2), lambda i: (0, 0)),
        pl.BlockSpec((1, _C), lambda i: (0, 0)),
    ],
    out_specs=[
        pl.BlockSpec((1, _L, _C), lambda i: (i, 0, 0)),
        pl.BlockSpec((1, _L, _C), lambda i: (i // _N, 0, 0)),
        pl.BlockSpec((4, _C), lambda i: (0, 0)),
    ],
    out_shape=[
        jax.ShapeDtypeStruct((_G, _L, _C), jnp.float32),
        jax.ShapeDtypeStruct((_B, _L, _C), jnp.float32),
        jax.ShapeDtypeStruct((4, _C), jnp.float32),
    ],
)


def _bn_coeffs_rows(st, gb_ref):
    inv_y = 1.0 / (_G * _L)
    inv_m = 1.0 / (_B * _L)
    mean_y = st[0:1] * inv_y
    var_y = st[1:2] * inv_y - mean_y * mean_y
    sc_y = gb_ref[0:1, 0:_C] * lax.rsqrt(var_y + 1e-5)
    sh_y = gb_ref[1:2, 0:_C] - sc_y * mean_y
    mean_m = st[2:3] * inv_m
    var_m = st[3:4] * inv_m - mean_m * mean_m
    sc_m = gb_ref[0:1, _C:2 * _C] * lax.rsqrt(var_m + 1e-5)
    sh_m = gb_ref[1:2, _C:2 * _C] - sc_m * mean_m
    return sc_y, sh_y, sc_m, sh_m


def _bnw_body(y_ref, xm_ref, st_ref, gb_ref, w_ref, o_ref, *, transpose_out):
    sc_y, sh_y, sc_m, sh_m = _bn_coeffs_rows(st_ref[...], gb_ref)
    yn = jnp.maximum(y_ref[0] * sc_y + sh_y, 0.0)
    mn = jnp.maximum(xm_ref[0] * sc_m + sh_m, 0.0)
    if transpose_out:
        o_ref[0] = (
            lax.dot_general(w_ref[0:_C], yn, (((0,), (1,)), ((), ())),
                            preferred_element_type=jnp.float32)
            + lax.dot_general(w_ref[_C:2 * _C], mn, (((0,), (1,)), ((), ())),
                              preferred_element_type=jnp.float32))
    else:
        o_ref[0] = (
            jnp.dot(yn, w_ref[0:_C], preferred_element_type=jnp.float32)
            + jnp.dot(mn, w_ref[_C:2 * _C], preferred_element_type=jnp.float32))


def _make_bnw_call(transpose_out):
    oshape = (_G, _C, _L) if transpose_out else (_G, _L, _C)
    oblock = (1, _C, _L) if transpose_out else (1, _L, _C)
    return pl.pallas_call(
        functools.partial(_bnw_body, transpose_out=transpose_out),
        grid=(_G,),
        in_specs=[
            pl.BlockSpec((1, _L, _C), lambda i: (i, 0, 0)),
            pl.BlockSpec((1, _L, _C), lambda i: (i // _N, 0, 0)),
            pl.BlockSpec((4, _C), lambda i: (0, 0)),
            pl.BlockSpec((2, 2 * _C), lambda i: (0, 0)),
            pl.BlockSpec((2 * _C, _C), lambda i: (0, 0)),
        ],
        out_specs=pl.BlockSpec(oblock, lambda i: (i, 0, 0)),
        out_shape=jax.ShapeDtypeStruct(oshape, jnp.float32),
    )


_bnw_call = _make_bnw_call(False)
_bnwt_call = _make_bnw_call(True)


def _convt_body(t_ref, c_ref, dr_ref, b_ref, y_ref, xm_ref, st_ref):
    i = pl.program_id(0)
    n = lax.rem(i, _N)
    t = t_ref[0]
    dis = dr_ref[0:1, :]
    nl = dr_ref[1:2, :]
    v = lax.dot_general(t * dis, c_ref[...], (((1,), (1,)), ((), ())),
                        preferred_element_type=jnp.float32)
    y = v * dis + t * nl + b_ref[...]
    y_ref[0] = y

    @pl.when(i == 0)
    def _():
        st_ref[...] = jnp.zeros((_C, 4), jnp.float32)

    st_ref[:, 0:1] = st_ref[:, 0:1] + jnp.sum(y, axis=1, keepdims=True)
    st_ref[:, 1:2] = st_ref[:, 1:2] + jnp.sum(y * y, axis=1, keepdims=True)

    @pl.when(n == 0)
    def _():
        xm_ref[0] = y

    @pl.when(n != 0)
    def _():
        xm_ref[0] = jnp.maximum(xm_ref[0], y)

    @pl.when(n == _N - 1)
    def _():
        m = xm_ref[0]
        st_ref[:, 2:3] = st_ref[:, 2:3] + jnp.sum(m, axis=1, keepdims=True)
        st_ref[:, 3:4] = st_ref[:, 3:4] + jnp.sum(m * m, axis=1, keepdims=True)


_convt_call = pl.pallas_call(
    _convt_body,
    grid=(_G,),
    in_specs=[
        pl.BlockSpec((1, _C, _L), lambda i: (i, 0, 0)),
        pl.BlockSpec((_L, _L), lambda i: (0, 0)),
        pl.BlockSpec((2, _L), lambda i: (0, 0)),
        pl.BlockSpec((_C, 1), lambda i: (0, 0)),
    ],
    out_specs=[
        pl.BlockSpec((1, _C, _L), lambda i: (i, 0, 0)),
        pl.BlockSpec((1, _C, _L), lambda i: (i // _N, 0, 0)),
        pl.BlockSpec((_C, 4), lambda i: (0, 0)),
    ],
    out_shape=[
        jax.ShapeDtypeStruct((_G, _C, _L), jnp.float32),
        jax.ShapeDtypeStruct((_B, _C, _L), jnp.float32),
        jax.ShapeDtypeStruct((_C, 4), jnp.float32),
    ],
)


def _final_body(y_ref, xm_ref, st_ref, gb_ref, o_ref):
    st = st_ref[...]
    inv_y = 1.0 / (_G * _L)
    inv_m = 1.0 / (_B * _L)
    mean_y = st[:, 0:1] * inv_y
    var_y = st[:, 1:2] * inv_y - mean_y * mean_y
    sc_y = gb_ref[0:_C, 0:1] * lax.rsqrt(var_y + 1e-5)
    sh_y = gb_ref[0:_C, 1:2] - sc_y * mean_y
    mean_m = st[:, 2:3] * inv_m
    var_m = st[:, 3:4] * inv_m - mean_m * mean_m
    sc_m = gb_ref[_C:2 * _C, 0:1] * lax.rsqrt(var_m + 1e-5)
    sh_m = gb_ref[_C:2 * _C, 1:2] - sc_m * mean_m
    o_ref[0, 0:_C, :] = jnp.maximum(y_ref[0] * sc_y + sh_y, 0.0)
    o_ref[0, _C:2 * _C, :] = jnp.maximum(xm_ref[0] * sc_m + sh_m, 0.0)


_final_call = pl.pallas_call(
    _final_body,
    grid=(_G,),
    in_specs=[
        pl.BlockSpec((1, _C, _L), lambda i: (i, 0, 0)),
        pl.BlockSpec((1, _C, _L), lambda i: (i // _N, 0, 0)),
        pl.BlockSpec((_C, 4), lambda i: (0, 0)),
        pl.BlockSpec((2 * _C, 2), lambda i: (0, 0)),
    ],
    out_specs=pl.BlockSpec((1, 2 * _C, _L), lambda i: (i, 0, 0)),
    out_shape=jax.ShapeDtypeStruct((_G, 2 * _C, _L), jnp.float32),
)


def kernel(x, edge_index, W1, b1, W2, b2, W3, b3, g1, be1, g2, be2, g3, be3):
    src = edge_index[0].astype(jnp.int32)
    dst = edge_index[1].astype(jnp.int32)
    cmat = _sc_count_call()(src, dst)
    dn_col, dn_row = _deg_call(cmat)

    t1 = _xw_call(x.reshape(_G, _C, _L), W1)
    y1, xm1, st1 = _conv_call(t1, cmat, dn_col, b1.reshape(1, _C))
    t2 = _bnw_call(y1, xm1, st1, jnp.stack([g1, be1]), W2)
    y2, xm2, st2 = _conv_call(t2, cmat, dn_col, b2.reshape(1, _C))
    t3 = _bnwt_call(y2, xm2, st2, jnp.stack([g2, be2]), W3)
    y3, xm3, st3 = _convt_call(t3, cmat, dn_row, b3.reshape(_C, 1))
    return _final_call(y3, xm3, st3, jnp.stack([g3, be3], axis=1))
